# baseline (speedup 1.0000x reference)
import functools

import jax
import jax.numpy as jnp
from jax import lax
from jax.experimental import pallas as pl
from jax.experimental.pallas import tpu as pltpu
from jax.experimental.pallas import tpu_sc as plsc

E = 16384
D = 64
NC = 2
NS = 16
L = 16
NW = NC * NS
BW = E // NW
CH = 128
NCH = BW // CH


def _sc_body(edges_hbm, table_hbm, out_hbm, idx_u, idx_v, rows_u, rows_v,
             tbuf, out_v, sem):
    c = lax.axis_index("c")
    s = lax.axis_index("s")
    wid = s * NC + c
    base = wid * BW

    for j in range(NCH):
        pltpu.sync_copy(edges_hbm.at[pl.ds(base + j * CH, CH)], idx_u.at[j])
        pltpu.sync_copy(edges_hbm.at[pl.ds(E + base + j * CH, CH)],
                        idx_v.at[j])

    copies = []
    for j in range(NCH):
        copies.append(pltpu.async_copy(
            table_hbm.at[idx_u.at[j]],
            rows_u.at[pl.ds(j * CH, CH)], sem))
        copies.append(pltpu.async_copy(
            table_hbm.at[idx_v.at[j]],
            rows_v.at[pl.ds(j * CH, CH)], sem))
    for cp in copies:
        cp.wait()

    lanes = lax.iota(jnp.int32, L)

    def group_body(g, carry):
        for ei in range(L):
            e = g * L + ei
            p = jnp.zeros((L,), jnp.float32)
            for c in range(D // L):
                u = rows_u[e, pl.ds(c * L, L)]
                v = rows_v[e, pl.ds(c * L, L)]
                p = p + u * v
            plsc.store_scatter(tbuf, [lanes * L + ei], p)
        acc = jnp.zeros((L,), jnp.float32)
        for k in range(L):
            acc = acc + tbuf[pl.ds(k * L, L)]
        sig = 1.0 / (1.0 + jnp.exp(-acc))
        out_v[pl.ds(g * L, L)] = sig
        return carry

    lax.fori_loop(0, BW // L, group_body, 0)

    pltpu.sync_copy(out_v, out_hbm.at[pl.ds(base, BW)])


@functools.partial(jax.jit, static_argnums=())
def _run(edges_flat, emb_table):
    mesh = plsc.VectorSubcoreMesh(core_axis_name="c", subcore_axis_name="s")
    f = pl.kernel(
        _sc_body,
        out_type=jax.ShapeDtypeStruct((E,), jnp.float32),
        mesh=mesh,
        scratch_types=[
            pltpu.VMEM((NCH, CH), jnp.int32),
            pltpu.VMEM((NCH, CH), jnp.int32),
            pltpu.VMEM((BW, D), jnp.float32),
            pltpu.VMEM((BW, D), jnp.float32),
            pltpu.VMEM((L * L,), jnp.float32),
            pltpu.VMEM((BW,), jnp.float32),
            pltpu.SemaphoreType.DMA,
        ],
        compiler_params=pltpu.CompilerParams(
            needs_layout_passes=False, use_tc_tiling_on_sc=False),
    )
    return f(edges_flat, emb_table)


def kernel(edges, emb_table):
    edges_flat = edges.reshape(-1).astype(jnp.int32)
    return _run(edges_flat, emb_table)
